# Initial kernel scaffold; baseline (speedup 1.0000x reference)
import jax
import jax.numpy as jnp
from jax.experimental import pallas as pl


def kernel(x, nodes, nbr1, nbr2, W1_self, W1_nbr, W2_self, W2_nbr, attn_w_pos, attn_b_pos, attn_w_nopos, attn_b_nopos, pe, Wout, bout):
    raise NotImplementedError("write your pallas kernel here")


# speedup vs baseline: 3.1034x; 3.1034x over previous
import functools

import jax
import jax.numpy as jnp
from jax import lax
from jax.experimental import pallas as pl
from jax.experimental.pallas import tpu as pltpu
from jax.experimental.pallas import tpu_sc as plsc

N, D, B, T = 100000, 128, 4096, 8
H1, H2 = 128, 64
S1, S2 = 5, 2
W_POS, W_NOPOS = 0.6, 0.4

NW = 32
CH = 128
G1_ROWS = S1 * T * B
H2_ROWS = S2 * S1 * T * B
G1_PW = G1_ROWS // NW
H2_PW = H2_ROWS // NW
H0_PW = B // NW


def _sc_gather(x, idx1, idx2, nodes):
    mesh = plsc.VectorSubcoreMesh(core_axis_name="c", subcore_axis_name="s")

    @functools.partial(
        pl.kernel,
        out_type=(
            jax.ShapeDtypeStruct((G1_ROWS, D), jnp.float32),
            jax.ShapeDtypeStruct((H2_ROWS, D), jnp.float32),
            jax.ShapeDtypeStruct((B, D), jnp.float32),
        ),
        mesh=mesh,
        scratch_types=[
            pltpu.VMEM((H2_PW,), jnp.int32),
            pltpu.VMEM((CH, D), jnp.float32),
            pltpu.SemaphoreType.DMA,
        ],
    )
    def k(x_hbm, idx1_hbm, idx2_hbm, nodes_hbm, g1_hbm, h2_hbm, h0_hbm,
          idx_v, buf, gsem):
        wid = lax.axis_index("s") * 2 + lax.axis_index("c")

        def gather_phase(idx_hbm, n_pw, out_hbm):
            base = wid * n_pw
            pltpu.sync_copy(idx_hbm.at[pl.ds(base, n_pw)],
                            idx_v.at[pl.ds(0, n_pw)])

            def body(c, _):
                pltpu.async_copy(
                    x_hbm.at[idx_v.at[pl.ds(c * CH, CH)]], buf, gsem).wait()
                pltpu.sync_copy(buf, out_hbm.at[pl.ds(base + c * CH, CH)])
                return 0

            lax.fori_loop(0, n_pw // CH, body, 0)

        gather_phase(idx1_hbm, G1_PW, g1_hbm)
        gather_phase(idx2_hbm, H2_PW, h2_hbm)
        gather_phase(nodes_hbm, H0_PW, h0_hbm)

    return k(x, idx1, idx2, nodes)


def _tc_dense(g1, h2, h0, w1s, w1n, w2s, w2n, awp, awn, pe, wout, bout):
    NB = 16
    BT = B // NB

    def body(g1r, h2r, h0r, w1sr, w1nr, w2sr, w2nr, awpr, awnr, per,
             woutr, boutr, outr, seq):
        t = pl.program_id(1)
        w1s_ = w1sr[...]
        w1n_ = w1nr[...]
        g = [g1r[s, 0] for s in range(S1)]
        agg0 = (g[0] + g[1] + g[2] + g[3] + g[4]) * (1.0 / S1)
        z0 = jnp.maximum(h0r[...] @ w1s_ + agg0 @ w1n_, 0.0)
        zsum = None
        for s in range(S1):
            a1 = (h2r[s, 0] + h2r[s + S1, 0]) * 0.5
            z1 = jnp.maximum(g[s] @ w1s_ + a1 @ w1n_, 0.0)
            zsum = z1 if zsum is None else zsum + z1
        agg2 = zsum * (1.0 / S1)
        z2 = jnp.maximum(z0 @ w2sr[...] + agg2 @ w2nr[...], 0.0)
        seq[pl.ds(t, 1)] = z2[None]

        @pl.when(t == T - 1)
        def _():
            sq = seq[...]

            def attn(s_, w_):
                sc_ = jnp.sum(s_ * w_[None, None, :], axis=-1, keepdims=True)
                m = jnp.max(sc_, axis=0, keepdims=True)
                e = jnp.exp(sc_ - m)
                wt = e / jnp.sum(e, axis=0, keepdims=True)
                return jnp.sum(s_ * wt, axis=0)

            pe_ = per[...]
            awp_ = awpr[...]
            awn_ = awnr[...]
            emb0 = (attn(sq + pe_[:, None, :], awp_[0]) * W_POS
                    + attn(sq, awn_[0]) * W_NOPOS)
            sq1 = jnp.stack([sq[0], sq[2], sq[4], sq[6]])
            emb1 = (attn(sq1 + pe_[0:4][:, None, :], awp_[1]) * W_POS
                    + attn(sq1, awn_[1]) * W_NOPOS)
            stacked = (emb0 + emb1) * 0.5
            outr[...] = stacked @ woutr[...] + boutr[...]

    return pl.pallas_call(
        body,
        grid=(NB, T),
        in_specs=[
            pl.BlockSpec((S1, 1, BT, D), lambda b, t: (0, t, b, 0)),
            pl.BlockSpec((S2 * S1, 1, BT, D), lambda b, t: (0, t, b, 0)),
            pl.BlockSpec((BT, D), lambda b, t: (b, 0)),
            pl.BlockSpec((D, H1), lambda b, t: (0, 0)),
            pl.BlockSpec((D, H1), lambda b, t: (0, 0)),
            pl.BlockSpec((H1, H2), lambda b, t: (0, 0)),
            pl.BlockSpec((H1, H2), lambda b, t: (0, 0)),
            pl.BlockSpec((2, H2), lambda b, t: (0, 0)),
            pl.BlockSpec((2, H2), lambda b, t: (0, 0)),
            pl.BlockSpec((T, H2), lambda b, t: (0, 0)),
            pl.BlockSpec((H2, D), lambda b, t: (0, 0)),
            pl.BlockSpec((1, D), lambda b, t: (0, 0)),
        ],
        out_specs=pl.BlockSpec((BT, D), lambda b, t: (b, 0)),
        out_shape=jax.ShapeDtypeStruct((B, D), jnp.float32),
        scratch_shapes=[pltpu.VMEM((T, BT, H2), jnp.float32)],
        compiler_params=pltpu.CompilerParams(
            dimension_semantics=("arbitrary", "arbitrary"),
        ),
    )(g1, h2, h0, w1s, w1n, w2s, w2n, awp, awn, pe, wout, bout)


def kernel(x, nodes, nbr1, nbr2, W1_self, W1_nbr, W2_self, W2_nbr,
           attn_w_pos, attn_b_pos, attn_w_nopos, attn_b_nopos, pe, Wout,
           bout):
    del attn_b_pos, attn_b_nopos
    nodes_i = nodes.astype(jnp.int32).reshape(B)
    idx1 = jnp.transpose(nbr1.astype(jnp.int32).reshape(T, B, S1),
                         (2, 0, 1)).reshape(G1_ROWS)
    idx2 = jnp.transpose(nbr2.astype(jnp.int32).reshape(T, B, S1, S2),
                         (3, 2, 0, 1)).reshape(H2_ROWS)
    g1, h2, h0 = _sc_gather(x, idx1, idx2, nodes_i)
    return _tc_dense(
        g1.reshape(S1, T, B, D), h2.reshape(S2 * S1, T, B, D), h0,
        W1_self, W1_nbr, W2_self, W2_nbr, attn_w_pos, attn_w_nopos, pe,
        Wout, bout.reshape(1, D))
